# baseline (speedup 1.0000x reference)
import functools

import jax
import jax.numpy as jnp
from jax import lax
from jax.experimental import pallas as pl
from jax.experimental.pallas import tpu as pltpu
from jax.experimental.pallas import tpu_sc as plsc

B = 16384
D = 16
SUP = 128 // D
VOC = 1000000

NC = 2
NS = 16
NW = NC * NS
CH = 128
GPW = B // (NW * CH)

TILES = (VOC + 127) // 128
TPW = -(-TILES // NW)
DROWS = TILES * D

BLK = 2048


CN = 8
CW = CN * CH
ORW = CN * D
VOCP = ((VOC + 127) // 128) * 128
NFULL = VOCP // CW
TAILT = (VOCP - NFULL * CW) // CH
NCHW = 30
NREM = NFULL - NW * NCHW


def _sc_repack(uid_tt, mid_tt):
  mesh = plsc.VectorSubcoreMesh(core_axis_name="c", subcore_axis_name="s")

  @functools.partial(
      pl.kernel,
      out_type=(
          jax.ShapeDtypeStruct((DROWS, 128), jnp.float32),
          jax.ShapeDtypeStruct((DROWS, 128), jnp.float32),
      ),
      mesh=mesh,
      scratch_types=[
          pltpu.VMEM((D, CW), jnp.float32),
          pltpu.VMEM((D, CW), jnp.float32),
          pltpu.VMEM((ORW, 128), jnp.float32),
          pltpu.VMEM((ORW, 128), jnp.float32),
          pltpu.SemaphoreType.DMA,
          pltpu.SemaphoreType.DMA,
          pltpu.SemaphoreType.DMA,
          pltpu.SemaphoreType.DMA,
      ],
      compiler_params=pltpu.CompilerParams(needs_layout_passes=False),
  )
  def body(uid_t, mid_t, uid_d, mid_d, ina, inb, outa, outb,
           ia_sem, ib_sem, oa_sem, ob_sem):
    wid = lax.axis_index("s") * NC + lax.axis_index("c")
    lanes = lax.iota(jnp.int32, 16)
    rbase = lax.shift_right_logical(lanes, 3)
    cbase = (lanes & 7) * D

    def repack(src_v, dst_v, ntiles):
      def tile_step(tt, carry):
        for cc in range(CH // D):
          ridx = tt * D + cc * 2 + rbase
          for d in range(D):
            v = src_v[d, pl.ds(tt * CH + cc * D, 16)]
            plsc.store_scatter(dst_v, [ridx, cbase + d], v)
        return carry
      lax.fori_loop(0, ntiles, tile_step, 0, unroll=2)

    for src, dst in ((uid_t, uid_d), (mid_t, mid_d)):
      def in_copy(g, buf, sem, src=src):
        return pltpu.make_async_copy(
            src.at[:, pl.ds(g * CW, CW)], buf, sem)

      def out_copy(g, buf, sem, dst=dst):
        return pltpu.make_async_copy(
            buf, dst.at[pl.ds(g * ORW, ORW)], sem)

      g0 = wid * NCHW
      in_copy(g0, ina, ia_sem).start()
      in_copy(g0 + 1, inb, ib_sem).start()

      def chunk_pair(i, carry):
        ge = g0 + 2 * i
        go = ge + 1
        in_copy(ge, ina, ia_sem).wait()
        repack(ina, outa, CN)

        @pl.when(i > 0)
        def _():
          out_copy(ge - 2, outa, oa_sem).wait()
        out_copy(ge, outa, oa_sem).start()

        @pl.when(i < NCHW // 2 - 1)
        def _():
          in_copy(ge + 2, ina, ia_sem).start()
        in_copy(go, inb, ib_sem).wait()
        repack(inb, outb, CN)

        @pl.when(i > 0)
        def _():
          out_copy(go - 2, outb, ob_sem).wait()
        out_copy(go, outb, ob_sem).start()

        @pl.when(i < NCHW // 2 - 1)
        def _():
          in_copy(go + 2, inb, ib_sem).start()
        return carry

      lax.fori_loop(0, NCHW // 2, chunk_pair, 0)
      out_copy(g0 + NCHW - 2, outa, oa_sem).wait()
      out_copy(g0 + NCHW - 1, outb, ob_sem).wait()

      @pl.when(wid < NREM)
      def _():
        g = NW * NCHW + wid
        in_copy(g, ina, ia_sem).start()
        in_copy(g, ina, ia_sem).wait()
        repack(ina, outa, CN)
        out_copy(g, outa, oa_sem).start()
        out_copy(g, outa, oa_sem).wait()

      @pl.when(wid == NREM)
      def _():
        g = jnp.int32(NFULL)
        tin = pltpu.make_async_copy(
            src.at[:, pl.ds(g * CW, TAILT * CH)],
            ina.at[:, pl.ds(0, TAILT * CH)], ia_sem)
        tin.start()
        tin.wait()
        repack(ina, outa, TAILT)
        tout = pltpu.make_async_copy(
            outa.at[pl.ds(0, TAILT * D)],
            dst.at[pl.ds(g * ORW, TAILT * D)], oa_sem)
        tout.start()
        tout.wait()

  return body(uid_tt, mid_tt)


def _sc_gather(uid_sup, mid_sup, uid_idx, mid_idx):
  mesh = plsc.VectorSubcoreMesh(core_axis_name="c", subcore_axis_name="s")

  @functools.partial(
      pl.kernel,
      out_type=(
          jax.ShapeDtypeStruct((NW, GPW, CH, 128), jnp.float32),
          jax.ShapeDtypeStruct((NW, GPW, CH, 128), jnp.float32),
      ),
      mesh=mesh,
      scratch_types=[
          pltpu.VMEM((GPW, CH), jnp.int32),
          pltpu.VMEM((GPW, CH), jnp.int32),
          pltpu.VMEM((GPW, CH, 128), jnp.float32),
          pltpu.SemaphoreType.DMA,
      ],
  )
  def body(uid_t, mid_t, uidx, midx, e_uid, e_mid,
           uidx_v, midx_v, rows, sem):
    wid = lax.axis_index("s") * NC + lax.axis_index("c")
    pltpu.sync_copy(uidx.at[wid], uidx_v)
    pltpu.sync_copy(midx.at[wid], midx_v)
    for src, idx_v, dst in ((uid_t, uidx_v, e_uid), (mid_t, midx_v, e_mid)):
      copies = [
          pltpu.async_copy(src.at[idx_v.at[j]], rows.at[j], sem)
          for j in range(GPW)
      ]
      for c in copies:
        c.wait()
      pltpu.sync_copy(rows, dst.at[wid])

  return body(uid_sup, mid_sup, uid_idx, mid_idx)


def _tc_math(su_uid, su_mid, sub_uid, sub_mid, gender2, age2, job2,
             gender_table, age_table, job_table, W1, b1r, W2r, b2r,
             Wsu, Wsm, bsr):
  def extract(sup, sub):
    acc = jnp.zeros((BLK, D), jnp.float32)
    for k in range(SUP):
      mask = (sub == k).astype(jnp.float32)
      acc = acc + mask * sup[:, k * D:(k + 1) * D]
    return acc

  def body(su_ref, sm_ref, bu_ref, bm_ref, g_ref, a_ref, j_ref,
           gt_ref, at_ref, jt_ref,
           w1_ref, b1_ref, w2_ref, b2_ref, wsu_ref, wsm_ref, bs_ref,
           out_ref, lam_ref):
    e_uid = extract(su_ref[...], bu_ref[...])
    e_mid = extract(sm_ref[...], bm_ref[...])
    w1_top = w1_ref[0:D, :]
    w1_bot = w1_ref[D:2 * D, :]
    m1 = jnp.dot(e_mid, w1_top, preferred_element_type=jnp.float32) + b1_ref[...]

    def att_branch(idx_ref, tab_ref, vocab):
      idx = idx_ref[...]
      iot = lax.broadcasted_iota(jnp.int32, (BLK, vocab), 1)
      oh = (idx == iot).astype(jnp.float32)
      e = jnp.dot(oh, tab_ref[...], preferred_element_type=jnp.float32)
      h = jnp.tanh(m1 + jnp.dot(e, w1_bot, preferred_element_type=jnp.float32))
      s = jnp.sum(h * w2_ref[...], axis=1, keepdims=True) + b2_ref[...]
      return e, s

    e_g, s_g = att_branch(g_ref, gt_ref, 2)
    e_a, s_a = att_branch(a_ref, at_ref, 7)
    e_j, s_j = att_branch(j_ref, jt_ref, 21)

    s = jnp.concatenate([s_g, s_a, s_j], axis=1)
    m = jnp.max(s, axis=1, keepdims=True)
    ex = jnp.exp(s - m)
    lam = ex / jnp.sum(ex, axis=1, keepdims=True)
    lam_ref[...] = lam

    fu = (lam[:, 0:1] * e_g + lam[:, 1:2] * e_a + lam[:, 2:3] * e_j + e_uid)
    out_ref[...] = (jnp.sum(fu * wsu_ref[...], axis=1, keepdims=True)
                    + jnp.sum(e_mid * wsm_ref[...], axis=1, keepdims=True)
                    + bs_ref[...])

  nblk = B // BLK
  full = lambda shape: pl.BlockSpec(shape, lambda i: tuple(0 for _ in shape))
  blk = lambda shape: pl.BlockSpec(shape, lambda i: (i,) + (0,) * (len(shape) - 1))
  return pl.pallas_call(
      body,
      grid=(nblk,),
      in_specs=[
          blk((BLK, 128)), blk((BLK, 128)),
          blk((BLK, 1)), blk((BLK, 1)),
          blk((BLK, 1)), blk((BLK, 1)), blk((BLK, 1)),
          full((2, D)), full((7, D)), full((21, D)),
          full((2 * D, D)), full((1, D)), full((1, D)), full((1, 1)),
          full((1, D)), full((1, D)), full((1, 1)),
      ],
      out_specs=[blk((BLK, 1)), blk((BLK, 3))],
      out_shape=[
          jax.ShapeDtypeStruct((B, 1), jnp.float32),
          jax.ShapeDtypeStruct((B, 3), jnp.float32),
      ],
  )(su_uid, su_mid, sub_uid, sub_mid, gender2, age2, job2,
    gender_table, age_table, job_table,
    W1, b1r, W2r, b2r, Wsu, Wsm, bsr)


def kernel(uid_table, gender_table, age_table, job_table, mid_table,
           W1, b1, W2, b2, W_svd, b_svd,
           uid, gender, age, job, mid):
  uid = uid.astype(jnp.int32)
  mid = mid.astype(jnp.int32)
  uid_dense, mid_dense = _sc_repack(
      jnp.swapaxes(uid_table, 0, 1), jnp.swapaxes(mid_table, 0, 1))
  su_uid, su_mid = _sc_gather(
      uid_dense, mid_dense,
      (uid // SUP).reshape(NW, GPW, CH), (mid // SUP).reshape(NW, GPW, CH))
  su_uid = su_uid.reshape(B, 128)
  su_mid = su_mid.reshape(B, 128)

  out, lam = _tc_math(
      su_uid, su_mid,
      (uid % SUP).reshape(B, 1), (mid % SUP).reshape(B, 1),
      gender.astype(jnp.int32).reshape(B, 1),
      age.astype(jnp.int32).reshape(B, 1),
      job.astype(jnp.int32).reshape(B, 1),
      gender_table, age_table, job_table,
      W1, b1.reshape(1, D), W2.reshape(1, D), b2.reshape(1, 1),
      W_svd[:D].reshape(1, D), W_svd[D:].reshape(1, D), b_svd.reshape(1, 1))
  return (out, lam.reshape(B, 3, 1))


# speedup vs baseline: 1.0945x; 1.0945x over previous
import functools

import jax
import jax.numpy as jnp
from jax import lax
from jax.experimental import pallas as pl
from jax.experimental.pallas import tpu as pltpu
from jax.experimental.pallas import tpu_sc as plsc

B = 16384
D = 16
SUP = 128 // D
VOC = 1000000

NC = 2
NS = 16
NW = NC * NS
CH = 128
GPW = B // (NW * CH)

TILES = (VOC + 127) // 128
TPW = -(-TILES // NW)
DROWS = TILES * D

BLK = 2048


CN = 8
CW = CN * CH
ORW = CN * D
VOCP = ((VOC + 127) // 128) * 128
NFULL = VOCP // CW
TAILT = (VOCP - NFULL * CW) // CH
NCHW = 30
NREM = NFULL - NW * NCHW


def _sc_repack(uid_tt, mid_tt):
  mesh = plsc.VectorSubcoreMesh(core_axis_name="c", subcore_axis_name="s")

  @functools.partial(
      pl.kernel,
      out_type=(
          jax.ShapeDtypeStruct((DROWS, 128), jnp.float32),
          jax.ShapeDtypeStruct((DROWS, 128), jnp.float32),
      ),
      mesh=mesh,
      scratch_types=[
          pltpu.VMEM((D, CW), jnp.float32),
          pltpu.VMEM((D, CW), jnp.float32),
          pltpu.VMEM((ORW, 128), jnp.float32),
          pltpu.VMEM((ORW, 128), jnp.float32),
          pltpu.SemaphoreType.DMA,
          pltpu.SemaphoreType.DMA,
          pltpu.SemaphoreType.DMA,
          pltpu.SemaphoreType.DMA,
      ],
      compiler_params=pltpu.CompilerParams(needs_layout_passes=False),
  )
  def body(uid_t, mid_t, uid_d, mid_d, ina, inb, outa, outb,
           ia_sem, ib_sem, oa_sem, ob_sem):
    wid = lax.axis_index("s") * NC + lax.axis_index("c")
    lanes = lax.iota(jnp.int32, 16)
    rbase = lax.shift_right_logical(lanes, 3)
    cbase = (lanes & 7) * D

    def repack(src_v, dst_v, ntiles):
      def tile_step(tt, carry):
        for cc in range(CH // D):
          ridx = tt * D + cc * 2 + rbase
          for d in range(D):
            v = src_v[d, pl.ds(tt * CH + cc * D, 16)]
            plsc.store_scatter(dst_v, [ridx, cbase + d], v)
        return carry
      lax.fori_loop(0, ntiles, tile_step, 0)

    for src, dst in ((uid_t, uid_d), (mid_t, mid_d)):
      def in_copy(g, buf, sem, src=src):
        return pltpu.make_async_copy(
            src.at[:, pl.ds(g * CW, CW)], buf, sem)

      def out_copy(g, buf, sem, dst=dst):
        return pltpu.make_async_copy(
            buf, dst.at[pl.ds(g * ORW, ORW)], sem)

      g0 = wid * NCHW
      in_copy(g0, ina, ia_sem).start()
      in_copy(g0 + 1, inb, ib_sem).start()

      def chunk_pair(i, carry):
        ge = g0 + 2 * i
        go = ge + 1
        in_copy(ge, ina, ia_sem).wait()
        repack(ina, outa, CN)

        @pl.when(i > 0)
        def _():
          out_copy(ge - 2, outa, oa_sem).wait()
        out_copy(ge, outa, oa_sem).start()

        @pl.when(i < NCHW // 2 - 1)
        def _():
          in_copy(ge + 2, ina, ia_sem).start()
        in_copy(go, inb, ib_sem).wait()
        repack(inb, outb, CN)

        @pl.when(i > 0)
        def _():
          out_copy(go - 2, outb, ob_sem).wait()
        out_copy(go, outb, ob_sem).start()

        @pl.when(i < NCHW // 2 - 1)
        def _():
          in_copy(go + 2, inb, ib_sem).start()
        return carry

      lax.fori_loop(0, NCHW // 2, chunk_pair, 0)
      out_copy(g0 + NCHW - 2, outa, oa_sem).wait()
      out_copy(g0 + NCHW - 1, outb, ob_sem).wait()

      @pl.when(wid < NREM)
      def _():
        g = NW * NCHW + wid
        in_copy(g, ina, ia_sem).start()
        in_copy(g, ina, ia_sem).wait()
        repack(ina, outa, CN)
        out_copy(g, outa, oa_sem).start()
        out_copy(g, outa, oa_sem).wait()

      @pl.when(wid == NREM)
      def _():
        g = jnp.int32(NFULL)
        tin = pltpu.make_async_copy(
            src.at[:, pl.ds(g * CW, TAILT * CH)],
            ina.at[:, pl.ds(0, TAILT * CH)], ia_sem)
        tin.start()
        tin.wait()
        repack(ina, outa, TAILT)
        tout = pltpu.make_async_copy(
            outa.at[pl.ds(0, TAILT * D)],
            dst.at[pl.ds(g * ORW, TAILT * D)], oa_sem)
        tout.start()
        tout.wait()

  return body(uid_tt, mid_tt)


def _sc_gather(uid_sup, mid_sup, uid_idx, mid_idx):
  mesh = plsc.VectorSubcoreMesh(core_axis_name="c", subcore_axis_name="s")

  @functools.partial(
      pl.kernel,
      out_type=(
          jax.ShapeDtypeStruct((NW, GPW, CH, 128), jnp.float32),
          jax.ShapeDtypeStruct((NW, GPW, CH, 128), jnp.float32),
      ),
      mesh=mesh,
      scratch_types=[
          pltpu.VMEM((GPW, CH), jnp.int32),
          pltpu.VMEM((GPW, CH), jnp.int32),
          pltpu.VMEM((GPW, CH, 128), jnp.float32),
          pltpu.SemaphoreType.DMA,
      ],
  )
  def body(uid_t, mid_t, uidx, midx, e_uid, e_mid,
           uidx_v, midx_v, rows, sem):
    wid = lax.axis_index("s") * NC + lax.axis_index("c")
    pltpu.sync_copy(uidx.at[wid], uidx_v)
    pltpu.sync_copy(midx.at[wid], midx_v)
    for src, idx_v, dst in ((uid_t, uidx_v, e_uid), (mid_t, midx_v, e_mid)):
      copies = [
          pltpu.async_copy(src.at[idx_v.at[j]], rows.at[j], sem)
          for j in range(GPW)
      ]
      for c in copies:
        c.wait()
      pltpu.sync_copy(rows, dst.at[wid])

  return body(uid_sup, mid_sup, uid_idx, mid_idx)


def _tc_math(su_uid, su_mid, aux, gender_table, age_table,
             job_table, W1, b1r, W2r, b2r, Wsu, Wsm, bsr):
  def body(su_ref, sm_ref, aux_ref,
           gt_ref, at_ref, jt_ref,
           w1_ref, b1_ref, w2_ref, b2_ref, wsu_ref, wsm_ref, bs_ref,
           out_ref, lam_ref):
    aux = aux_ref[...]
    lane_grp = lax.broadcasted_iota(jnp.int32, (BLK, SUP * D), 1) // D

    def extract(sup, sub):
      sel = jnp.where(lane_grp == sub, sup, 0.0)
      acc = sel[:, 0:D]
      for k in range(1, SUP):
        acc = acc + sel[:, k * D:(k + 1) * D]
      return acc

    e_uid = extract(su_ref[...], aux[:, 0:1])
    e_mid = extract(sm_ref[...], aux[:, 1:2])
    w1_top = w1_ref[0:D, :]
    w1_bot = w1_ref[D:2 * D, :]
    m1 = jnp.dot(e_mid, w1_top, preferred_element_type=jnp.float32) + b1_ref[...]

    def att_branch(idx, tab_ref, vocab):
      iot = lax.broadcasted_iota(jnp.int32, (BLK, vocab), 1)
      oh = (idx == iot).astype(jnp.float32)
      e = jnp.dot(oh, tab_ref[...], preferred_element_type=jnp.float32)
      h = jnp.tanh(m1 + jnp.dot(e, w1_bot, preferred_element_type=jnp.float32))
      s = jnp.sum(h * w2_ref[...], axis=1, keepdims=True) + b2_ref[...]
      return e, s

    e_g, s_g = att_branch(aux[:, 2:3], gt_ref, 2)
    e_a, s_a = att_branch(aux[:, 3:4], at_ref, 7)
    e_j, s_j = att_branch(aux[:, 4:5], jt_ref, 21)

    s = jnp.concatenate([s_g, s_a, s_j], axis=1)
    m = jnp.max(s, axis=1, keepdims=True)
    ex = jnp.exp(s - m)
    lam = ex / jnp.sum(ex, axis=1, keepdims=True)
    lam_ref[...] = lam

    fu = (lam[:, 0:1] * e_g + lam[:, 1:2] * e_a + lam[:, 2:3] * e_j + e_uid)
    out_ref[...] = (jnp.sum(fu * wsu_ref[...], axis=1, keepdims=True)
                    + jnp.sum(e_mid * wsm_ref[...], axis=1, keepdims=True)
                    + bs_ref[...])

  nblk = B // BLK
  full = lambda shape: pl.BlockSpec(shape, lambda i: tuple(0 for _ in shape))
  blk = lambda shape: pl.BlockSpec(shape, lambda i: (i,) + (0,) * (len(shape) - 1))
  return pl.pallas_call(
      body,
      grid=(nblk,),
      in_specs=[
          blk((BLK, 128)), blk((BLK, 128)), blk((BLK, 5)),
          full((2, D)), full((7, D)), full((21, D)),
          full((2 * D, D)), full((1, D)), full((1, D)), full((1, 1)),
          full((1, D)), full((1, D)), full((1, 1)),
      ],
      out_specs=[blk((BLK, 1)), blk((BLK, 3))],
      out_shape=[
          jax.ShapeDtypeStruct((B, 1), jnp.float32),
          jax.ShapeDtypeStruct((B, 3), jnp.float32),
      ],
  )(su_uid, su_mid, aux, gender_table, age_table, job_table,
    W1, b1r, W2r, b2r, Wsu, Wsm, bsr)


def kernel(uid_table, gender_table, age_table, job_table, mid_table,
           W1, b1, W2, b2, W_svd, b_svd,
           uid, gender, age, job, mid):
  uid = uid.astype(jnp.int32)
  mid = mid.astype(jnp.int32)
  uid_dense, mid_dense = _sc_repack(
      jnp.swapaxes(uid_table, 0, 1), jnp.swapaxes(mid_table, 0, 1))
  su_uid, su_mid = _sc_gather(
      uid_dense, mid_dense,
      (uid // SUP).reshape(NW, GPW, CH), (mid // SUP).reshape(NW, GPW, CH))
  su_uid = su_uid.reshape(B, 128)
  su_mid = su_mid.reshape(B, 128)

  aux = jnp.stack(
      [uid % SUP, mid % SUP, gender.astype(jnp.int32),
       age.astype(jnp.int32), job.astype(jnp.int32)], axis=1)

  out, lam = _tc_math(
      su_uid, su_mid, aux,
      gender_table, age_table, job_table,
      W1, b1.reshape(1, D), W2.reshape(1, D), b2.reshape(1, 1),
      W_svd[:D].reshape(1, D), W_svd[D:].reshape(1, D), b_svd.reshape(1, 1))
  return (out, lam.reshape(B, 3, 1))
